# baseline (speedup 1.0000x reference)
import functools

import jax
import jax.numpy as jnp
from jax import lax
from jax.experimental import pallas as pl
from jax.experimental.pallas import tpu as pltpu
from jax.experimental.pallas import tpu_sc as plsc

CODES = 1024
SPECIAL = 2
QUANT = 8
WIDTH = 384
LENGTH = 2250
BATCH = 32
MAIN_ROWS = QUANT * CODES
TROWS = MAIN_ROWS + SPECIAL

CHUNK = 128
PAD_N = 2304
NCHUNK = PAD_N // CHUNK
SUB = 32
NSUB = CHUNK // SUB
TAIL = LENGTH - (NCHUNK - 1) * CHUNK
TAIL_FULL = TAIL // SUB
TAIL_REST = TAIL - TAIL_FULL * SUB
LANES = 16
BUILD_SUB = 32
BUILD_STEPS = MAIN_ROWS // LANES // BUILD_SUB


def _make_embed_sum():
    mesh = plsc.VectorSubcoreMesh(core_axis_name="c", subcore_axis_name="s")
    num_cores = mesh.num_cores

    @functools.partial(
        pl.kernel,
        out_type=[
            jax.ShapeDtypeStruct((BATCH, LENGTH, WIDTH), jnp.float32),
            jax.ShapeDtypeStruct((TROWS, WIDTH), jnp.bfloat16),
        ],
        mesh=mesh,
        scratch_types=[
            pltpu.VMEM((QUANT, LENGTH), jnp.int32),
            pltpu.VMEM((QUANT, PAD_N), jnp.int32),
            pltpu.VMEM((2, CHUNK, WIDTH), jnp.bfloat16),
            pltpu.VMEM((2, SUB, WIDTH), jnp.float32),
            pltpu.SemaphoreType.DMA,
            pltpu.SemaphoreType.DMA,
        ],
        compiler_params=pltpu.CompilerParams(
            use_tc_tiling_on_sc=False, needs_layout_passes=False
        ),
    )
    def embed_sum(
        toks_hbm, mains_hbm, special_hbm, out_hbm, table_hbm,
        tok_v, idx_f, acc, fbuf, sem_g, sem_o,
    ):
        cid = lax.axis_index("c")
        sid = lax.axis_index("s")
        wid = sid * num_cores + cid

        iota = lax.iota(jnp.int32, LANES)
        iota2 = 2 * iota
        mask_hi = jnp.full((LANES,), -65536, jnp.int32)
        mask_lo = jnp.full((LANES,), 0xFFFF, jnp.int32)
        one = jnp.full((LANES,), 1, jnp.int32)
        rbias = jnp.full((LANES,), 0x7FFF, jnp.int32)

        def rne_bf16_words(x):
            r = lax.bitwise_and(lax.shift_right_logical(x, 16), one)
            return x + rbias + r

        def build_step(k, carry):
            r0 = sid * (BUILD_SUB * BUILD_STEPS) + k * BUILD_SUB
            fp = lax.rem(k, 2)
            pltpu.sync_copy(mains_hbm.at[pl.ds(r0, BUILD_SUB)], fbuf.at[fp])

            def row_body(p, carry2):
                fvec = jnp.full((LANES,), fp, jnp.int32)
                pvec = jnp.full((LANES,), p, jnp.int32)
                for j in range(WIDTH // (2 * LANES)):
                    cols = iota2 + (j * 2 * LANES)
                    ev = plsc.bitcast(
                        plsc.load_gather(fbuf, [fvec, pvec, cols]), jnp.int32
                    )
                    od = plsc.bitcast(
                        plsc.load_gather(fbuf, [fvec, pvec, cols + 1]),
                        jnp.int32,
                    )
                    w = lax.bitwise_or(
                        lax.bitwise_and(
                            lax.shift_right_logical(rne_bf16_words(ev), 16),
                            mask_lo,
                        ),
                        lax.bitwise_and(rne_bf16_words(od), mask_hi),
                    )
                    acc[0, p, pl.ds(j * 2 * LANES, 2 * LANES)] = plsc.bitcast(
                        w, jnp.bfloat16
                    )
                return carry2

            lax.fori_loop(0, BUILD_SUB, row_body, 0)
            pltpu.sync_copy(
                acc.at[0, pl.ds(0, BUILD_SUB)], table_hbm.at[pl.ds(r0, BUILD_SUB)]
            )
            return carry

        lax.fori_loop(0, BUILD_STEPS, build_step, 0)

        @pl.when(sid == 0)
        def _():
            pltpu.sync_copy(special_hbm, fbuf.at[0, pl.ds(0, SPECIAL)])

            def srow_body(p, carry2):
                fvec = jnp.full((LANES,), 0, jnp.int32)
                pvec = jnp.full((LANES,), p, jnp.int32)
                for j in range(WIDTH // (2 * LANES)):
                    cols = iota2 + (j * 2 * LANES)
                    ev = plsc.bitcast(
                        plsc.load_gather(fbuf, [fvec, pvec, cols]), jnp.int32
                    )
                    od = plsc.bitcast(
                        plsc.load_gather(fbuf, [fvec, pvec, cols + 1]),
                        jnp.int32,
                    )
                    w = lax.bitwise_or(
                        lax.bitwise_and(
                            lax.shift_right_logical(rne_bf16_words(ev), 16),
                            mask_lo,
                        ),
                        lax.bitwise_and(rne_bf16_words(od), mask_hi),
                    )
                    acc[0, p, pl.ds(j * 2 * LANES, 2 * LANES)] = plsc.bitcast(
                        w, jnp.bfloat16
                    )
                return carry2

            lax.fori_loop(0, SPECIAL, srow_body, 0)
            pltpu.sync_copy(
                acc.at[0, pl.ds(0, SPECIAL)],
                table_hbm.at[pl.ds(MAIN_ROWS, SPECIAL)],
            )

        plsc.subcore_barrier()

        pltpu.sync_copy(toks_hbm.at[wid], tok_v)

        def flat_ids(t, q):
            t_c = jnp.clip(t, 0, CODES + SPECIAL - 1)
            return jnp.where(
                t_c < CODES, t_c + q * CODES, t_c + (MAIN_ROWS - CODES)
            )

        def fix_body(g, carry):
            sl = pl.ds(g * LANES, LANES)
            for q in range(QUANT):
                idx_f[q, sl] = flat_ids(tok_v[q, sl], q)
            return carry

        lax.fori_loop(0, LENGTH // LANES, fix_body, 0)

        last = (LENGTH // LANES) * LANES
        maxcol = jnp.full((LANES,), LENGTH - 1, jnp.int32)
        for q in range(QUANT):
            qvec = jnp.full((LANES,), q, jnp.int32)
            cols = jnp.minimum(last + iota, maxcol)
            t = plsc.load_gather(tok_v, [qvec, cols])
            idx_f[q, pl.ds(last, LANES)] = flat_ids(t, q)
            for g in range(last // LANES + 1, PAD_N // LANES):
                idx_f[q, pl.ds(g * LANES, LANES)] = jnp.zeros(
                    (LANES,), jnp.int32
                )

        zeros_bf = jnp.zeros((2 * LANES,), jnp.bfloat16)

        def zero_par(par):
            def zero_body(p, carry2):
                for j in range(WIDTH // (2 * LANES)):
                    acc[par, p, pl.ds(j * 2 * LANES, 2 * LANES)] = zeros_bf
                return carry2

            lax.fori_loop(0, CHUNK, zero_body, 0)

        def issue_gathers(c, par):
            base = pl.multiple_of(c * CHUNK, CHUNK)
            for q in range(QUANT):
                pltpu.async_copy(
                    table_hbm.at[idx_f.at[q, pl.ds(base, CHUNK)]],
                    acc.at[par],
                    sem_g,
                    add=True,
                )

        def wait_gathers():
            for _ in range(QUANT):
                pltpu.make_async_copy(
                    table_hbm.at[idx_f.at[0, pl.ds(0, CHUNK)]],
                    acc.at[0],
                    sem_g,
                ).wait()

        def convert_sub(par, base, s, rows):
            fpar = s % 2

            def row_body(p, carry2):
                pvec = jnp.full((LANES,), p, jnp.int32)
                fvec = jnp.full((LANES,), fpar, jnp.int32)
                for j in range(WIDTH // (2 * LANES)):
                    w = plsc.bitcast(
                        acc[par, s * SUB + p, pl.ds(j * 2 * LANES, 2 * LANES)],
                        jnp.int32,
                    )
                    even = plsc.bitcast(lax.shift_left(w, 16), jnp.float32)
                    odd = plsc.bitcast(
                        lax.bitwise_and(w, mask_hi), jnp.float32
                    )
                    cols = iota2 + (j * 2 * LANES)
                    plsc.store_scatter(fbuf, [fvec, pvec, cols], even)
                    plsc.store_scatter(fbuf, [fvec, pvec, cols + 1], odd)
                return carry2

            lax.fori_loop(0, rows, row_body, 0)
            pltpu.async_copy(
                fbuf.at[fpar, pl.ds(0, rows)],
                out_hbm.at[wid, pl.ds(base + s * SUB, rows)],
                sem_o,
            )

        def drain_out(rows):
            pltpu.make_async_copy(
                fbuf.at[0, pl.ds(0, rows)],
                out_hbm.at[wid, pl.ds(0, rows)],
                sem_o,
            ).wait()

        zero_par(0)

        def chunk_body(c, carry):
            par = lax.rem(c, 2)
            ppar = lax.rem(c + 1, 2)

            issue_gathers(c, par)

            @pl.when(c >= 1)
            def _():
                prev_base = pl.multiple_of((c - 1) * CHUNK, CHUNK)
                for s in range(NSUB):
                    if s < 2:
                        @pl.when(c >= 2)
                        def _():
                            drain_out(SUB)
                    else:
                        drain_out(SUB)
                    convert_sub(ppar, prev_base, s, SUB)

            zero_par(ppar)
            wait_gathers()
            return carry

        lax.fori_loop(0, NCHUNK, chunk_body, 0)

        last_par = (NCHUNK - 1) % 2
        last_base = (NCHUNK - 1) * CHUNK
        for s in range(TAIL_FULL):
            drain_out(SUB)
            convert_sub(last_par, last_base, s, SUB)
        drain_out(SUB)
        convert_sub(last_par, last_base, TAIL_FULL, TAIL_REST)

        drain_out(SUB)
        drain_out(TAIL_REST)

    return embed_sum


def kernel(toks, xenc, mains, special):
    del xenc
    toks32 = toks.astype(jnp.int32)
    out, _ = _make_embed_sum()(
        toks32, mains.reshape(MAIN_ROWS, WIDTH), special
    )
    return out


# speedup vs baseline: 1.6014x; 1.6014x over previous
import functools

import jax
import jax.numpy as jnp
from jax import lax
from jax.experimental import pallas as pl
from jax.experimental.pallas import tpu as pltpu
from jax.experimental.pallas import tpu_sc as plsc

CODES = 1024
SPECIAL = 2
QUANT = 8
WIDTH = 384
LENGTH = 2250
BATCH = 32
ROWS = CODES + SPECIAL

CHUNK = 128
PAD_N = 2304
NCHUNK = PAD_N // CHUNK
SUB = 32
NSUB = CHUNK // SUB
TAIL = LENGTH - (NCHUNK - 1) * CHUNK
TAIL_FULL = TAIL // SUB
TAIL_REST = TAIL - TAIL_FULL * SUB
LANES = 16


def _make_embed_sum():
    mesh = plsc.VectorSubcoreMesh(core_axis_name="c", subcore_axis_name="s")
    num_cores = mesh.num_cores

    @functools.partial(
        pl.kernel,
        out_type=jax.ShapeDtypeStruct((BATCH, LENGTH, WIDTH), jnp.float32),
        mesh=mesh,
        scratch_types=[
            pltpu.VMEM((QUANT, PAD_N), jnp.int32),
            pltpu.VMEM((2, CHUNK, WIDTH), jnp.bfloat16),
            pltpu.VMEM((2, SUB, WIDTH), jnp.float32),
            pltpu.SemaphoreType.DMA,
            pltpu.SemaphoreType.DMA,
        ],
        compiler_params=pltpu.CompilerParams(
            use_tc_tiling_on_sc=False, needs_layout_passes=False
        ),
    )
    def embed_sum(toks_hbm, table_hbm, out_hbm, idx_v, acc, fbuf, sem_g, sem_o):
        wid = lax.axis_index("s") * num_cores + lax.axis_index("c")

        pltpu.sync_copy(toks_hbm.at[wid], idx_v)

        def fix_body(g, carry):
            sl = pl.ds(g * LANES, LANES)
            for q in range(QUANT):
                t = idx_v[q, sl]
                idx_v[q, sl] = jnp.clip(t, 0, ROWS - 1) + q * ROWS
            return carry

        lax.fori_loop(0, PAD_N // LANES, fix_body, 0)

        zeros_bf = jnp.zeros((2 * LANES,), jnp.bfloat16)
        iota2 = 2 * jax.lax.iota(jnp.int32, LANES)
        mask_hi = jnp.full((LANES,), -65536, jnp.int32)

        def convert_sub(par, base, s, rows):
            fpar = s % 2

            def row_body(p, carry2):
                for j in range(WIDTH // (2 * LANES)):
                    w = plsc.bitcast(
                        acc[par, s * SUB + p, pl.ds(j * 2 * LANES, 2 * LANES)],
                        jnp.int32,
                    )
                    even = plsc.bitcast(lax.shift_left(w, 16), jnp.float32)
                    odd = plsc.bitcast(
                        lax.bitwise_and(w, mask_hi), jnp.float32
                    )
                    cols = iota2 + (j * 2 * LANES)
                    pvec = jnp.full((LANES,), p, jnp.int32)
                    fvec = jnp.full((LANES,), fpar, jnp.int32)
                    plsc.store_scatter(fbuf, [fvec, pvec, cols], even)
                    plsc.store_scatter(fbuf, [fvec, pvec, cols + 1], odd)
                return carry2

            lax.fori_loop(0, rows, row_body, 0)
            pltpu.async_copy(
                fbuf.at[fpar, pl.ds(0, rows)],
                out_hbm.at[wid, pl.ds(base + s * SUB, rows)],
                sem_o,
            )

        def drain_out(rows):
            pltpu.make_async_copy(
                fbuf.at[0, pl.ds(0, rows)],
                out_hbm.at[wid, pl.ds(0, rows)],
                sem_o,
            ).wait()

        def zero_par(par):
            def zero_body(p, carry2):
                for j in range(WIDTH // (2 * LANES)):
                    acc[par, p, pl.ds(j * 2 * LANES, 2 * LANES)] = zeros_bf
                return carry2

            lax.fori_loop(0, CHUNK, zero_body, 0)

        zero_par(0)

        def chunk_body(c, carry):
            base = pl.multiple_of(c * CHUNK, CHUNK)
            par = lax.rem(c, 2)
            ppar = lax.rem(c + 1, 2)

            descs = [
                pltpu.async_copy(
                    table_hbm.at[idx_v.at[q, pl.ds(base, CHUNK)]],
                    acc.at[par],
                    sem_g,
                    add=True,
                )
                for q in range(QUANT)
            ]

            @pl.when(c >= 1)
            def _():
                prev_base = base - CHUNK
                for s in range(NSUB):
                    if s < 2:
                        @pl.when(c >= 2)
                        def _():
                            drain_out(SUB)
                    else:
                        drain_out(SUB)
                    convert_sub(ppar, prev_base, s, SUB)

            zero_par(ppar)

            for d in descs:
                d.wait()

            return carry

        lax.fori_loop(0, NCHUNK, chunk_body, 0)

        last_par = (NCHUNK - 1) % 2
        last_base = (NCHUNK - 1) * CHUNK
        for s in range(TAIL_FULL):
            drain_out(SUB)
            convert_sub(last_par, last_base, s, SUB)
        drain_out(SUB)
        convert_sub(last_par, last_base, TAIL_FULL, TAIL_REST)

        drain_out(SUB)
        drain_out(TAIL_REST)

    return embed_sum


def kernel(toks, xenc, mains, special):
    del xenc
    toks32 = jnp.pad(
        toks.astype(jnp.int32), ((0, 0), (0, 0), (0, PAD_N - LENGTH))
    )
    table = (
        jnp.concatenate(
            [mains, jnp.broadcast_to(special[None], (QUANT, SPECIAL, WIDTH))],
            axis=1,
        )
        .reshape(QUANT * ROWS, WIDTH)
        .astype(jnp.bfloat16)
    )
    return _make_embed_sum()(toks32, table)
